# baseline (speedup 1.0000x reference)
import math

import jax
import jax.numpy as jnp
from jax.experimental import pallas as pl
from jax.experimental.pallas import tpu as pltpu

D_H = 2048
E = 64
NBP = 4
R = D_H // (2 * NBP)
SCALE = 1.0 / math.sqrt(2048.0)


def _qdot(ctx, wq):
    return jax.lax.dot_general(
        ctx, wq, (((1,), (1,)), ((), ())), preferred_element_type=jnp.float32)


def _tdot(q, wk):
    return jax.lax.dot_general(
        q, wk, (((1,), (0,)), ((), ())), preferred_element_type=jnp.float32)


def _body(ctx_ref, wqa_ref, wqb_ref, wka_ref, wkb_ref, keys_ref, out_ref,
          t_ref, qa_ref, qb_ref):
    i = pl.program_id(0)

    @pl.when(i == 0)
    def _init():
        t_ref[...] = jnp.zeros_like(t_ref)

    @pl.when(i > 0)
    def _fold():
        t_ref[...] += (_tdot(qa_ref[...], wka_ref[...]) +
                       _tdot(qb_ref[...], wkb_ref[...]))

    @pl.when(i < NBP)
    def _proj():
        ctx = ctx_ref[...]
        qa_ref[...] = _qdot(ctx, wqa_ref[...])
        qb_ref[...] = _qdot(ctx, wqb_ref[...])

    @pl.when(i == NBP)
    def _fin():
        s = jax.lax.dot_general(
            t_ref[...], keys_ref[...], (((1,), (1,)), ((), ())),
            preferred_element_type=jnp.float32) * SCALE
        m = jnp.max(s, axis=-1, keepdims=True)
        ex = jnp.exp(s - m)
        out_ref[...] = ex / jnp.sum(ex, axis=-1, keepdims=True)


def kernel(expert_outputs, context, keys, Wq, Wk):
    del expert_outputs
    ctx2 = context.reshape(1, D_H)
    last = NBP - 1
    gate = pl.pallas_call(
        _body,
        grid=(NBP + 1,),
        in_specs=[
            pl.BlockSpec((1, D_H), lambda i: (0, 0)),
            pl.BlockSpec((R, D_H), lambda i: (jnp.minimum(i, last), 0)),
            pl.BlockSpec((R, D_H), lambda i: (jnp.minimum(i, last) + NBP, 0)),
            pl.BlockSpec((R, D_H), lambda i: (jnp.maximum(i - 1, 0), 0)),
            pl.BlockSpec((R, D_H), lambda i: (jnp.maximum(i - 1, 0) + NBP, 0)),
            pl.BlockSpec((E, D_H), lambda i: (0, 0)),
        ],
        out_specs=pl.BlockSpec((1, E), lambda i: (0, 0)),
        out_shape=jax.ShapeDtypeStruct((1, E), jnp.float32),
        scratch_shapes=[
            pltpu.VMEM((1, D_H), jnp.float32),
            pltpu.VMEM((1, R), jnp.float32),
            pltpu.VMEM((1, R), jnp.float32),
        ],
        compiler_params=pltpu.CompilerParams(
            dimension_semantics=("arbitrary",),
        ),
    )(ctx2, Wq, Wq, Wk, Wk, keys)
    return gate.reshape(E)


# speedup vs baseline: 1.0259x; 1.0259x over previous
import math

import jax
import jax.numpy as jnp
from jax.experimental import pallas as pl
from jax.experimental.pallas import tpu as pltpu

D_H = 2048
E = 64
NB = 2
NS = 2
R = D_H // (NS * NB)
SCALE = 1.0 / math.sqrt(2048.0)


def _chain(ctx, wq, wk):
    q = jax.lax.dot_general(
        ctx, wq, (((1,), (1,)), ((), ())), preferred_element_type=jnp.float32)
    return jax.lax.dot_general(
        q, wk, (((1,), (0,)), ((), ())), preferred_element_type=jnp.float32)


def _body(*refs):
    ctx_ref = refs[0]
    wq_refs = refs[1:1 + NS]
    wk_refs = refs[1 + NS:1 + 2 * NS]
    keys_ref = refs[1 + 2 * NS]
    out_ref = refs[2 + 2 * NS]
    t_ref = refs[3 + 2 * NS]
    i = pl.program_id(0)

    @pl.when(i == 0)
    def _init():
        t_ref[...] = jnp.zeros_like(t_ref)

    ctx = ctx_ref[...]
    acc = _chain(ctx, wq_refs[0][...], wk_refs[0][...])
    for s in range(1, NS):
        acc += _chain(ctx, wq_refs[s][...], wk_refs[s][...])
    t_ref[...] += acc

    @pl.when(i == NB - 1)
    def _fin():
        s = jax.lax.dot_general(
            t_ref[...], keys_ref[...], (((1,), (1,)), ((), ())),
            preferred_element_type=jnp.float32) * SCALE
        m = jnp.max(s, axis=-1, keepdims=True)
        ex = jnp.exp(s - m)
        out_ref[...] = ex / jnp.sum(ex, axis=-1, keepdims=True)


def _mk_spec(s):
    return pl.BlockSpec((R, D_H), lambda i, s=s: (i + s * NB, 0))


def kernel(expert_outputs, context, keys, Wq, Wk):
    del expert_outputs
    ctx2 = context.reshape(1, D_H)
    gate = pl.pallas_call(
        _body,
        grid=(NB,),
        in_specs=(
            [pl.BlockSpec((1, D_H), lambda i: (0, 0))]
            + [_mk_spec(s) for s in range(NS)]
            + [_mk_spec(s) for s in range(NS)]
            + [pl.BlockSpec((E, D_H), lambda i: (0, 0))]
        ),
        out_specs=pl.BlockSpec((1, E), lambda i: (0, 0)),
        out_shape=jax.ShapeDtypeStruct((1, E), jnp.float32),
        scratch_shapes=[pltpu.VMEM((1, D_H), jnp.float32)],
        compiler_params=pltpu.CompilerParams(
            dimension_semantics=("arbitrary",),
        ),
    )(ctx2, *([Wq] * NS), *([Wk] * NS), keys)
    return gate.reshape(E)


# speedup vs baseline: 1.0281x; 1.0021x over previous
import math

import jax
import jax.numpy as jnp
from jax.experimental import pallas as pl
from jax.experimental.pallas import tpu as pltpu

D_H = 2048
E = 64
NB = 2
NS = 2
R = D_H // (NS * NB)
SCALE = 1.0 / math.sqrt(2048.0)


def _chain(ctx, wq, wk):
    q = jax.lax.dot_general(
        ctx, wq, (((1,), (1,)), ((), ())), preferred_element_type=jnp.float32)
    return jax.lax.dot_general(
        q, wk, (((1,), (0,)), ((), ())), preferred_element_type=jnp.float32)


def _body(*refs):
    ctx_ref = refs[0]
    wq_refs = refs[1:1 + NS]
    wk_refs = refs[1 + NS:1 + 2 * NS]
    keys_ref = refs[1 + 2 * NS]
    out_ref = refs[2 + 2 * NS]
    t_ref = refs[3 + 2 * NS]
    i = pl.program_id(0)

    @pl.when(i == 0)
    def _init():
        t_ref[...] = jnp.zeros_like(t_ref)

    ctx = ctx_ref[...]
    acc = _chain(ctx, wq_refs[0][...], wk_refs[0][...])
    for s in range(1, NS):
        acc += _chain(ctx, wq_refs[s][...], wk_refs[s][...])
    t_ref[...] += acc

    @pl.when(i == NB - 1)
    def _fin():
        s = jax.lax.dot_general(
            t_ref[...], keys_ref[...], (((1,), (1,)), ((), ())),
            preferred_element_type=jnp.float32) * SCALE
        m = jnp.max(s, axis=-1, keepdims=True)
        ex = jnp.exp(s - m)
        out_ref[...] = ex / jnp.sum(ex, axis=-1, keepdims=True)


def _mk_spec(s):
    return pl.BlockSpec((R, D_H), lambda i, s=s: (i * NS + s, 0))


def kernel(expert_outputs, context, keys, Wq, Wk):
    del expert_outputs
    ctx2 = context.reshape(1, D_H)
    gate = pl.pallas_call(
        _body,
        grid=(NB,),
        in_specs=(
            [pl.BlockSpec((1, D_H), lambda i: (0, 0))]
            + [_mk_spec(s) for s in range(NS)]
            + [_mk_spec(s) for s in range(NS)]
            + [pl.BlockSpec((E, D_H), lambda i: (0, 0))]
        ),
        out_specs=pl.BlockSpec((1, E), lambda i: (0, 0)),
        out_shape=jax.ShapeDtypeStruct((1, E), jnp.float32),
        scratch_shapes=[pltpu.VMEM((1, D_H), jnp.float32)],
        compiler_params=pltpu.CompilerParams(
            dimension_semantics=("arbitrary",),
        ),
    )(ctx2, *([Wq] * NS), *([Wk] * NS), keys)
    return gate.reshape(E)
